# baseline (speedup 1.0000x reference)
import functools

import jax
import jax.numpy as jnp
from jax import lax
from jax.experimental import pallas as pl
from jax.experimental.pallas import tpu as pltpu
from jax.experimental.pallas import tpu_sc as plsc

_GAMMA = 12.0
_B = 16384
_D = 128
_NW = 32
_BPW = _B // _NW
_C = 64
_SIZES = (_C,) * 8
_OFFS = tuple(range(0, _BPW, _C))
_NCHUNK = len(_SIZES)
_NSEG = _D // 16
_PSTRIDE = 17
_NSLOT = 6

_mesh = plsc.VectorSubcoreMesh(core_axis_name="c", subcore_axis_name="s")


@functools.partial(
    pl.kernel,
    mesh=_mesh,
    out_type=jax.ShapeDtypeStruct((_B,), jnp.float32),
    compiler_params=pltpu.CompilerParams(needs_layout_passes=False),
    scratch_types=[
        pltpu.VMEM((_NSLOT, _C), jnp.int32),
        pltpu.VMEM((_NSLOT, _C), jnp.int32),
        pltpu.VMEM((_NSLOT, _C), jnp.int32),
        pltpu.VMEM((_NSLOT, _C, _D), jnp.float32),
        pltpu.VMEM((_NSLOT, _C, _D), jnp.float32),
        pltpu.VMEM((16 * _PSTRIDE,), jnp.float32),
        pltpu.VMEM((_BPW,), jnp.float32),
        [pltpu.SemaphoreType.DMA] * _NSLOT,
        [pltpu.SemaphoreType.DMA] * _NSLOT,
        [pltpu.SemaphoreType.DMA] * _NSLOT,
    ],
)
def _kge_score(hidx_hbm, ridx_hbm, tidx_hbm, ent_hbm, rel_hbm, out_hbm,
               hi_v, ri_v, ti_v, x_v, t_v, p_v, score_v,
               isems, rsems, gsems):
    wid = lax.axis_index("s") * 2 + lax.axis_index("c")
    base = pl.multiple_of(wid * _BPW, _BPW)
    iota16 = lax.iota(jnp.int32, 16)

    idx_cp = {}
    rel_cp = {}
    ht_cp = {}

    def start_idx(ci):
        slot = ci % _NSLOT
        sz = _SIZES[ci]
        off = pl.multiple_of(base + _OFFS[ci], 8)
        idx_cp[ci] = [
            pltpu.async_copy(src.at[pl.ds(off, sz)],
                             dst.at[slot, pl.ds(0, sz)], isems[slot])
            for src, dst in ((hidx_hbm, hi_v), (ridx_hbm, ri_v),
                             (tidx_hbm, ti_v))
        ]

    def start_rel(ci):
        slot = ci % _NSLOT
        sz = _SIZES[ci]
        for cp in idx_cp[ci]:
            cp.wait()
        rel_cp[ci] = pltpu.async_copy(
            rel_hbm.at[ri_v.at[slot, pl.ds(0, sz)]],
            x_v.at[slot, pl.ds(0, sz)], rsems[slot])

    def start_head_tail(ci):
        slot = ci % _NSLOT
        sz = _SIZES[ci]
        rel_cp[ci].wait()
        ht_cp[ci] = [
            pltpu.async_copy(
                ent_hbm.at[hi_v.at[slot, pl.ds(0, sz)]],
                x_v.at[slot, pl.ds(0, sz)], gsems[slot], add=True),
            pltpu.async_copy(
                ent_hbm.at[ti_v.at[slot, pl.ds(0, sz)]],
                t_v.at[slot, pl.ds(0, sz)], gsems[slot]),
        ]

    def compute(ci):
        slot = ci % _NSLOT

        def group_body(g, carry):
            for rr in range(16):
                row = g * 16 + rr
                acc = jnp.zeros((16,), jnp.float32)
                for j in range(_NSEG):
                    xseg = x_v[slot, row, pl.ds(j * 16, 16)]
                    tseg = t_v[slot, row, pl.ds(j * 16, 16)]
                    acc = acc + jnp.abs(xseg - tseg)
                plsc.store_scatter(p_v, [iota16 * _PSTRIDE + rr], acc)
            tot = jnp.zeros((16,), jnp.float32)
            for j in range(16):
                tot = tot + plsc.load_gather(p_v, [iota16 + j * _PSTRIDE])
            score_v[pl.ds(_OFFS[ci] + g * 16, 16)] = _GAMMA - tot
            return carry

        lax.fori_loop(0, _SIZES[ci] // 16, group_body, 0)

    start_idx(0)
    start_idx(1)
    start_idx(2)
    start_rel(0)
    start_head_tail(0)
    start_rel(1)
    start_rel(2)
    start_idx(3)
    start_idx(4)
    for ci in range(_NCHUNK):
        for cp in ht_cp[ci]:
            cp.wait()
        if ci + 1 < _NCHUNK:
            start_head_tail(ci + 1)
        if ci + 3 < _NCHUNK:
            start_rel(ci + 3)
        if ci + 5 < _NCHUNK:
            start_idx(ci + 5)
        compute(ci)

    pltpu.sync_copy(score_v, out_hbm.at[pl.ds(base, _BPW)])


def kernel(sample, ent_emb, rel_emb, time_emb):
    del time_emb
    hidx = sample[:, 0]
    ridx = sample[:, 1]
    tidx = sample[:, 2]
    score = _kge_score(hidx, ridx, tidx, ent_emb, rel_emb)
    return score[:, None]


# speedup vs baseline: 1.1418x; 1.1418x over previous
import functools

import jax
import jax.numpy as jnp
from jax import lax
from jax.experimental import pallas as pl
from jax.experimental.pallas import tpu as pltpu
from jax.experimental.pallas import tpu_sc as plsc

_GAMMA = 12.0
_B = 16384
_D = 128
_NW = 32
_BPW = _B // _NW
_C = 128
_SIZES = (32, 96, 128, 128, 96, 32)
_OFFS = (0, 32, 128, 256, 384, 480)
_NCHUNK = len(_SIZES)
_NSEG = _D // 16
_PSTRIDE = 17
_NSLOT = 3

_mesh = plsc.VectorSubcoreMesh(core_axis_name="c", subcore_axis_name="s")


@functools.partial(
    pl.kernel,
    mesh=_mesh,
    out_type=jax.ShapeDtypeStruct((_B,), jnp.float32),
    compiler_params=pltpu.CompilerParams(needs_layout_passes=False),
    scratch_types=[
        pltpu.VMEM((_NSLOT, _C), jnp.int32),
        pltpu.VMEM((_NSLOT, _C), jnp.int32),
        pltpu.VMEM((_NSLOT, _C), jnp.int32),
        pltpu.VMEM((_NSLOT, _C, _D), jnp.float32),
        pltpu.VMEM((_NSLOT, _C, _D), jnp.float32),
        pltpu.VMEM((_C // 16, 16 * _PSTRIDE), jnp.float32),
        pltpu.VMEM((_BPW,), jnp.float32),
        [pltpu.SemaphoreType.DMA] * _NSLOT,
        [pltpu.SemaphoreType.DMA] * _NSLOT,
        [pltpu.SemaphoreType.DMA] * _NSLOT,
    ],
)
def _kge_score(hidx_hbm, ridx_hbm, tidx_hbm, ent_hbm, rel_hbm, out_hbm,
               hi_v, ri_v, ti_v, x_v, t_v, p_v, score_v,
               isems, rsems, gsems):
    wid = lax.axis_index("s") * 2 + lax.axis_index("c")
    base = pl.multiple_of(wid * _BPW, _BPW)
    iota16 = lax.iota(jnp.int32, 16)

    idx_cp = {}
    rel_cp = {}
    ht_cp = {}

    def start_idx(ci):
        slot = ci % _NSLOT
        sz = _SIZES[ci]
        off = pl.multiple_of(base + _OFFS[ci], 8)
        idx_cp[ci] = [
            pltpu.async_copy(src.at[pl.ds(off, sz)],
                             dst.at[slot, pl.ds(0, sz)], isems[slot])
            for src, dst in ((hidx_hbm, hi_v), (ridx_hbm, ri_v),
                             (tidx_hbm, ti_v))
        ]

    def start_rel(ci):
        slot = ci % _NSLOT
        sz = _SIZES[ci]
        for cp in idx_cp[ci]:
            cp.wait()
        rel_cp[ci] = pltpu.async_copy(
            rel_hbm.at[ri_v.at[slot, pl.ds(0, sz)]],
            x_v.at[slot, pl.ds(0, sz)], rsems[slot])

    def start_head_tail(ci):
        slot = ci % _NSLOT
        sz = _SIZES[ci]
        rel_cp[ci].wait()
        ht_cp[ci] = [
            pltpu.async_copy(
                ent_hbm.at[hi_v.at[slot, pl.ds(0, sz)]],
                x_v.at[slot, pl.ds(0, sz)], gsems[slot], add=True),
            pltpu.async_copy(
                ent_hbm.at[ti_v.at[slot, pl.ds(0, sz)]],
                t_v.at[slot, pl.ds(0, sz)], gsems[slot]),
        ]

    def compute(ci):
        slot = ci % _NSLOT
        sz = _SIZES[ci]

        def row_body(i, carry):
            for k in range(4):
                row = i * 4 + k
                g = row >> 4
                rr = row & 15
                acc = jnp.zeros((16,), jnp.float32)
                for j in range(_NSEG):
                    xseg = x_v[slot, row, pl.ds(j * 16, 16)]
                    tseg = t_v[slot, row, pl.ds(j * 16, 16)]
                    acc = acc + jnp.abs(xseg - tseg)
                plsc.store_scatter(
                    p_v, [jnp.full((16,), g, jnp.int32),
                          iota16 * _PSTRIDE + rr], acc)
            return carry

        def group_body(g, carry):
            g_vec = jnp.full((16,), g, jnp.int32)
            tot = jnp.zeros((16,), jnp.float32)
            for j in range(16):
                tot = tot + plsc.load_gather(
                    p_v, [g_vec, iota16 + j * _PSTRIDE])
            score_v[pl.ds(_OFFS[ci] + g * 16, 16)] = _GAMMA - tot
            return carry

        lax.fori_loop(0, sz // 4, row_body, 0)
        lax.fori_loop(0, sz // 16, group_body, 0)

    start_idx(0)
    start_idx(1)
    start_idx(2)
    start_rel(0)
    start_head_tail(0)
    start_rel(1)
    start_rel(2)
    start_idx(3)
    start_idx(4)
    for ci in range(_NCHUNK):
        for cp in ht_cp[ci]:
            cp.wait()
        if ci + 1 < _NCHUNK:
            start_head_tail(ci + 1)
        if ci + 3 < _NCHUNK:
            start_rel(ci + 3)
        if ci + 5 < _NCHUNK:
            start_idx(ci + 5)
        compute(ci)

    pltpu.sync_copy(score_v, out_hbm.at[pl.ds(base, _BPW)])


def kernel(sample, ent_emb, rel_emb, time_emb):
    del time_emb
    hidx = sample[:, 0]
    ridx = sample[:, 1]
    tidx = sample[:, 2]
    score = _kge_score(hidx, ridx, tidx, ent_emb, rel_emb)
    return score[:, None]


# speedup vs baseline: 1.1535x; 1.0103x over previous
import functools

import jax
import jax.numpy as jnp
from jax import lax
from jax.experimental import pallas as pl
from jax.experimental.pallas import tpu as pltpu
from jax.experimental.pallas import tpu_sc as plsc

_GAMMA = 12.0
_B = 16384
_D = 128
_NW = 32
_BPW = _B // _NW
_C = 128
_SIZES = (32, 96, 128, 128, 96, 32)
_OFFS = (0, 32, 128, 256, 384, 480)
_NCHUNK = len(_SIZES)
_NSEG = _D // 16
_PSTRIDE = 17
_NSLOT = 3

_mesh = plsc.VectorSubcoreMesh(core_axis_name="c", subcore_axis_name="s")


@functools.partial(
    pl.kernel,
    mesh=_mesh,
    out_type=jax.ShapeDtypeStruct((_B,), jnp.float32),
    compiler_params=pltpu.CompilerParams(needs_layout_passes=False),
    scratch_types=[
        pltpu.VMEM((_NSLOT, _C), jnp.int32),
        pltpu.VMEM((_NSLOT, _C), jnp.int32),
        pltpu.VMEM((_NSLOT, _C), jnp.int32),
        pltpu.VMEM((_NSLOT, _C, _D), jnp.float32),
        pltpu.VMEM((_NSLOT, _C, _D), jnp.float32),
        pltpu.VMEM(((_C // 16) * 16 * _PSTRIDE,), jnp.float32),
        pltpu.VMEM((_BPW,), jnp.float32),
        [pltpu.SemaphoreType.DMA] * _NSLOT,
        [pltpu.SemaphoreType.DMA] * _NSLOT,
        [pltpu.SemaphoreType.DMA] * _NSLOT,
    ],
)
def _kge_score(hidx_hbm, ridx_hbm, tidx_hbm, ent_hbm, rel_hbm, out_hbm,
               hi_v, ri_v, ti_v, x_v, t_v, p_v, score_v,
               isems, rsems, gsems):
    wid = lax.axis_index("s") * 2 + lax.axis_index("c")
    base = pl.multiple_of(wid * _BPW, _BPW)
    iota16 = lax.iota(jnp.int32, 16)

    idx_cp = {}
    rel_cp = {}
    ht_cp = {}

    def start_idx(ci):
        slot = ci % _NSLOT
        sz = _SIZES[ci]
        off = pl.multiple_of(base + _OFFS[ci], 8)
        idx_cp[ci] = [
            pltpu.async_copy(src.at[pl.ds(off, sz)],
                             dst.at[slot, pl.ds(0, sz)], isems[slot])
            for src, dst in ((hidx_hbm, hi_v), (ridx_hbm, ri_v),
                             (tidx_hbm, ti_v))
        ]

    def start_rel(ci):
        slot = ci % _NSLOT
        sz = _SIZES[ci]
        for cp in idx_cp[ci]:
            cp.wait()
        rel_cp[ci] = pltpu.async_copy(
            rel_hbm.at[ri_v.at[slot, pl.ds(0, sz)]],
            x_v.at[slot, pl.ds(0, sz)], rsems[slot])

    def start_head_tail(ci):
        slot = ci % _NSLOT
        sz = _SIZES[ci]
        rel_cp[ci].wait()
        ht_cp[ci] = [
            pltpu.async_copy(
                ent_hbm.at[hi_v.at[slot, pl.ds(0, sz)]],
                x_v.at[slot, pl.ds(0, sz)], gsems[slot], add=True),
            pltpu.async_copy(
                ent_hbm.at[ti_v.at[slot, pl.ds(0, sz)]],
                t_v.at[slot, pl.ds(0, sz)], gsems[slot]),
        ]

    def compute(ci):
        slot = ci % _NSLOT
        sz = _SIZES[ci]

        def row_body(i, carry):
            for k in range(4):
                row = i * 4 + k
                g = row >> 4
                rr = row & 15
                acc = jnp.zeros((16,), jnp.float32)
                for j in range(_NSEG):
                    xseg = x_v[slot, row, pl.ds(j * 16, 16)]
                    tseg = t_v[slot, row, pl.ds(j * 16, 16)]
                    acc = acc + jnp.abs(xseg - tseg)
                pbase = g * (16 * _PSTRIDE) + rr
                plsc.store_scatter(p_v, [iota16 * _PSTRIDE + pbase], acc)
            return carry

        def group_body(g, carry):
            gbase = g * (16 * _PSTRIDE)
            tot = jnp.zeros((16,), jnp.float32)
            for j in range(16):
                tot = tot + plsc.load_gather(
                    p_v, [iota16 + (gbase + j * _PSTRIDE)])
            score_v[pl.ds(_OFFS[ci] + g * 16, 16)] = _GAMMA - tot
            return carry

        lax.fori_loop(0, sz // 4, row_body, 0)
        lax.fori_loop(0, sz // 16, group_body, 0)

    start_idx(0)
    start_idx(1)
    start_rel(0)
    start_head_tail(0)
    start_rel(1)
    start_idx(2)
    for ci in range(_NCHUNK):
        for cp in ht_cp[ci]:
            cp.wait()
        if ci + 1 < _NCHUNK:
            start_head_tail(ci + 1)
        if ci + 2 < _NCHUNK:
            start_rel(ci + 2)
        if ci + 3 < _NCHUNK:
            start_idx(ci + 3)
        compute(ci)

    pltpu.sync_copy(score_v, out_hbm.at[pl.ds(base, _BPW)])


def kernel(sample, ent_emb, rel_emb, time_emb):
    del time_emb
    hidx = sample[:, 0]
    ridx = sample[:, 1]
    tidx = sample[:, 2]
    score = _kge_score(hidx, ridx, tidx, ent_emb, rel_emb)
    return score[:, None]
